# baseline (speedup 1.0000x reference)
import jax
import jax.numpy as jnp
from jax.experimental import pallas as pl
from jax.experimental.pallas import tpu as pltpu


NUM_ROWS = 100000
EMBED = 64
BLOCK_SUB = 64


def _copy_kernel(w_ref, o_ref):
    o_ref[...] = w_ref[...]


def kernel(x, W):
    del x
    Wt = W.T
    out_t = pl.pallas_call(
        _copy_kernel,
        grid=(EMBED // BLOCK_SUB,),
        in_specs=[pl.BlockSpec((BLOCK_SUB, NUM_ROWS), lambda i: (i, 0))],
        out_specs=pl.BlockSpec((BLOCK_SUB, NUM_ROWS), lambda i: (i, 0)),
        out_shape=jax.ShapeDtypeStruct((EMBED, NUM_ROWS), jnp.float32),
    )(Wt)
    return out_t.T


# speedup vs baseline: 1.1536x; 1.1536x over previous
import jax
import jax.numpy as jnp
from jax.experimental import pallas as pl
from jax.experimental.pallas import tpu as pltpu


NUM_ROWS = 100000
EMBED = 64
BLOCK_SUB = 32


def _copy_kernel(w_ref, o_ref):
    o_ref[...] = w_ref[...]


def kernel(x, W):
    del x
    Wt = W.T
    out_t = pl.pallas_call(
        _copy_kernel,
        grid=(EMBED // BLOCK_SUB,),
        in_specs=[pl.BlockSpec((BLOCK_SUB, NUM_ROWS), lambda i: (i, 0))],
        out_specs=pl.BlockSpec((BLOCK_SUB, NUM_ROWS), lambda i: (i, 0)),
        out_shape=jax.ShapeDtypeStruct((EMBED, NUM_ROWS), jnp.float32),
    )(Wt)
    return out_t.T
